# Initial kernel scaffold; baseline (speedup 1.0000x reference)
import jax
import jax.numpy as jnp
from jax.experimental import pallas as pl


def kernel(x, batch, params):
    raise NotImplementedError("write your pallas kernel here")


# speedup vs baseline: 13.0195x; 13.0195x over previous
import functools

import jax
import jax.numpy as jnp
from jax.experimental import pallas as pl
from jax.experimental.pallas import tpu as pltpu

N = 10000
B = 10
NPG = N // B
K = 7
DSH = 32
PROP = 64
SDIM = 3
IN_DIM = 9
OUT_DIM = 31


def _elu(v):
    return jnp.where(v > 0, v, jnp.exp(v) - 1.0)


def _gravnet_body(x_ref, *refs):
    out_ref = refs[-1]
    wrefs = list(refs[:-1])
    it = iter(wrefs)

    def nxt():
        return next(it)[...]

    input_w = nxt()
    xb = x_ref[...]
    cur = jnp.dot(xb, input_w, preferred_element_type=jnp.float32)

    iota_j = jax.lax.broadcasted_iota(jnp.int32, (NPG, NPG), 1)
    eye3 = (jax.lax.broadcasted_iota(jnp.int32, (SDIM, SDIM), 0)
            == jax.lax.broadcasted_iota(jnp.int32, (SDIM, SDIM), 1)
            ).astype(jnp.float32)

    feats = []
    for i in range(4):
        pre_w1 = nxt(); pre_b1 = nxt(); pre_w2 = nxt(); pre_b2 = nxt()
        lin_s_w = nxt(); lin_s_b = nxt(); lin_h_w = nxt(); lin_h_b = nxt()
        w_z = nxt(); w_mean = nxt(); w_max = nxt(); lin_out_b = nxt()
        post_wc = nxt(); post_ws = nxt(); post_b1 = nxt()
        post_w2 = nxt(); post_b2 = nxt()
        ge_wm = nxt(); ge_wn = nxt(); ge_wx = nxt(); ge_wz = nxt()
        out_b = nxt()

        z = _elu(jnp.dot(cur, pre_w1, preferred_element_type=jnp.float32) + pre_b1)
        z = _elu(jnp.dot(z, pre_w2, preferred_element_type=jnp.float32) + pre_b2)
        s = jnp.dot(z, lin_s_w, preferred_element_type=jnp.float32) + lin_s_b
        hf = jnp.dot(z, lin_h_w, preferred_element_type=jnp.float32) + lin_h_b

        sT = jax.lax.dot_general(eye3, s, (((1,), (1,)), ((), ())),
                                 preferred_element_type=jnp.float32)
        d2 = jnp.zeros((NPG, NPG), jnp.float32)
        for d in range(SDIM):
            diff = s[:, d:d + 1] - sT[d:d + 1, :]
            d2 = d2 + diff * diff

        def sel_step(_, carry):
            d2m, accsum, accmax = carry
            dmin = jnp.min(d2m, axis=1, keepdims=True)
            ismin = d2m == dmin
            idx = jnp.min(jnp.where(ismin, iota_j, jnp.int32(2**31 - 1)),
                          axis=1, keepdims=True)
            selb = iota_j == idx
            g = jnp.dot(selb.astype(jnp.float32), hf,
                        preferred_element_type=jnp.float32)
            msg = g * jnp.exp(-10.0 * dmin)
            accsum = accsum + msg
            accmax = jnp.maximum(accmax, msg)
            d2m = jnp.where(selb, jnp.float32(jnp.inf), d2m)
            return d2m, accsum, accmax

        _, accsum, accmax = jax.lax.fori_loop(
            0, K, sel_step,
            (d2, jnp.zeros((NPG, PROP), jnp.float32),
             jnp.full((NPG, PROP), -jnp.inf, jnp.float32)))

        conv = (jnp.dot(z, w_z, preferred_element_type=jnp.float32)
                + jnp.dot(accsum * (1.0 / K), w_mean,
                          preferred_element_type=jnp.float32)
                + jnp.dot(accmax, w_max, preferred_element_type=jnp.float32)
                + lin_out_b)
        zc = _elu(jnp.dot(conv, post_wc, preferred_element_type=jnp.float32)
                  + jnp.dot(s, post_ws, preferred_element_type=jnp.float32)
                  + post_b1)
        zc = _elu(jnp.dot(zc, post_w2, preferred_element_type=jnp.float32)
                  + post_b2)

        mean_f = jnp.mean(zc, axis=0, keepdims=True)
        min_f = jnp.min(zc, axis=0, keepdims=True)
        max_f = jnp.max(zc, axis=0, keepdims=True)
        zc = _elu(jnp.dot(mean_f, ge_wm, preferred_element_type=jnp.float32)
                  + jnp.dot(min_f, ge_wn, preferred_element_type=jnp.float32)
                  + jnp.dot(max_f, ge_wx, preferred_element_type=jnp.float32)
                  + jnp.dot(zc, ge_wz, preferred_element_type=jnp.float32)
                  + out_b)
        feats.append(zc)
        cur = zc

    h = jnp.zeros((NPG, 64), jnp.float32)
    for i in range(4):
        h = h + jnp.dot(feats[i], nxt(), preferred_element_type=jnp.float32)
    h = _elu(h + nxt())
    for _ in range(3):
        h = _elu(jnp.dot(h, nxt(), preferred_element_type=jnp.float32) + nxt())
    h = _elu(jnp.dot(h, nxt(), preferred_element_type=jnp.float32) + nxt())
    h = _elu(jnp.dot(h, nxt(), preferred_element_type=jnp.float32) + nxt())
    h = jnp.dot(h, nxt(), preferred_element_type=jnp.float32) + nxt()
    oc = jnp.dot(h, nxt(), preferred_element_type=jnp.float32) + nxt()
    ob = jnp.dot(h, nxt(), preferred_element_type=jnp.float32) + nxt()
    out_ref[...] = jnp.concatenate([oc, ob], axis=1)


def _row(b):
    return b.reshape(1, -1)


def _flatten_params(params):
    ws = [params['input_w']]
    for i in range(4):
        p = params['blocks'][i]
        lw = p['lin_out_w']
        pw = p['post_w1']
        ow = p['out_w']
        ws += [
            p['pre_w1'], _row(p['pre_b1']), p['pre_w2'], _row(p['pre_b2']),
            p['lin_s_w'], _row(p['lin_s_b']), p['lin_h_w'], _row(p['lin_h_b']),
            lw[:DSH], lw[DSH:DSH + PROP], lw[DSH + PROP:], _row(p['lin_out_b']),
            pw[:DSH], pw[DSH:], _row(p['post_b1']),
            p['post_w2'], _row(p['post_b2']),
            ow[:DSH], ow[DSH:2 * DSH], ow[2 * DSH:3 * DSH], ow[3 * DSH:],
            _row(p['out_b']),
        ]
    pg0_w, pg0_b = params['postgn'][0]
    ws += [pg0_w[:DSH], pg0_w[DSH:2 * DSH], pg0_w[2 * DSH:3 * DSH],
           pg0_w[3 * DSH:], _row(pg0_b)]
    for i in range(1, 4):
        w, b = params['postgn'][i]
        ws += [w, _row(b)]
    for k in ('out1', 'out2', 'out3', 'clust', 'beta'):
        w, b = params[k]
        ws += [w, _row(b)]
    return ws


@jax.jit
def kernel(x, batch, params):
    ws = _flatten_params(params)
    in_specs = [pl.BlockSpec((NPG, IN_DIM), lambda e: (e, 0))]
    for w in ws:
        in_specs.append(pl.BlockSpec(w.shape, lambda e: (0, 0)))
    return pl.pallas_call(
        _gravnet_body,
        grid=(B,),
        in_specs=in_specs,
        out_specs=pl.BlockSpec((NPG, OUT_DIM), lambda e: (e, 0)),
        out_shape=jax.ShapeDtypeStruct((N, OUT_DIM), jnp.float32),
        compiler_params=pltpu.CompilerParams(
            dimension_semantics=("parallel",)),
    )(x, *ws)


# speedup vs baseline: 13.2437x; 1.0172x over previous
import functools

import jax
import jax.numpy as jnp
from jax.experimental import pallas as pl
from jax.experimental.pallas import tpu as pltpu

N = 10000
B = 10
NPG = N // B
K = 7
DSH = 32
PROP = 64
SDIM = 3
IN_DIM = 9
OUT_DIM = 31


def _elu(v):
    return jnp.where(v > 0, v, jnp.exp(v) - 1.0)


def _gravnet_body(x_ref, *refs):
    out_ref = refs[-1]
    wrefs = list(refs[:-1])
    it = iter(wrefs)

    def nxt():
        return next(it)[...]

    input_w = nxt()
    xb = x_ref[...]
    cur = jnp.dot(xb, input_w, preferred_element_type=jnp.float32)

    eye3 =(jax.lax.broadcasted_iota(jnp.int32, (SDIM, SDIM), 0)
            == jax.lax.broadcasted_iota(jnp.int32, (SDIM, SDIM), 1)
            ).astype(jnp.float32)

    feats = []
    for i in range(4):
        pre_w1 = nxt(); pre_b1 = nxt(); pre_w2 = nxt(); pre_b2 = nxt()
        lin_s_w = nxt(); lin_s_b = nxt(); lin_h_w = nxt(); lin_h_b = nxt()
        w_z = nxt(); w_mean = nxt(); w_max = nxt(); lin_out_b = nxt()
        post_wc = nxt(); post_ws = nxt(); post_b1 = nxt()
        post_w2 = nxt(); post_b2 = nxt()
        ge_wm = nxt(); ge_wn = nxt(); ge_wx = nxt(); ge_wz = nxt()
        out_b = nxt()

        z = _elu(jnp.dot(cur, pre_w1, preferred_element_type=jnp.float32) + pre_b1)
        z = _elu(jnp.dot(z, pre_w2, preferred_element_type=jnp.float32) + pre_b2)
        s = jnp.dot(z, lin_s_w, preferred_element_type=jnp.float32) + lin_s_b
        hf = jnp.dot(z, lin_h_w, preferred_element_type=jnp.float32) + lin_h_b

        sT = jax.lax.dot_general(eye3, s, (((1,), (1,)), ((), ())),
                                 preferred_element_type=jnp.float32)
        d2 = jnp.zeros((NPG, NPG), jnp.float32)
        for d in range(SDIM):
            diff = s[:, d:d + 1] - sT[d:d + 1, :]
            d2 = d2 + diff * diff

        hf_hi = hf.astype(jnp.bfloat16)
        hf_lo = (hf - hf_hi.astype(jnp.float32)).astype(jnp.bfloat16)

        def sel_step(_, carry):
            d2m, accsum, accmax = carry
            dmin = jnp.min(d2m, axis=1, keepdims=True)
            selb = d2m == dmin
            selbf = selb.astype(jnp.bfloat16)
            g = (jnp.dot(selbf, hf_hi, preferred_element_type=jnp.float32)
                 + jnp.dot(selbf, hf_lo, preferred_element_type=jnp.float32))
            msg = g * jnp.exp(-10.0 * dmin)
            accsum = accsum + msg
            accmax = jnp.maximum(accmax, msg)
            d2m = jnp.where(selb, jnp.float32(jnp.inf), d2m)
            return d2m, accsum, accmax

        _, accsum, accmax = jax.lax.fori_loop(
            0, K, sel_step,
            (d2, jnp.zeros((NPG, PROP), jnp.float32),
             jnp.full((NPG, PROP), -jnp.inf, jnp.float32)))

        conv = (jnp.dot(z, w_z, preferred_element_type=jnp.float32)
                + jnp.dot(accsum * (1.0 / K), w_mean,
                          preferred_element_type=jnp.float32)
                + jnp.dot(accmax, w_max, preferred_element_type=jnp.float32)
                + lin_out_b)
        zc = _elu(jnp.dot(conv, post_wc, preferred_element_type=jnp.float32)
                  + jnp.dot(s, post_ws, preferred_element_type=jnp.float32)
                  + post_b1)
        zc = _elu(jnp.dot(zc, post_w2, preferred_element_type=jnp.float32)
                  + post_b2)

        mean_f = jnp.mean(zc, axis=0, keepdims=True)
        min_f = jnp.min(zc, axis=0, keepdims=True)
        max_f = jnp.max(zc, axis=0, keepdims=True)
        zc = _elu(jnp.dot(mean_f, ge_wm, preferred_element_type=jnp.float32)
                  + jnp.dot(min_f, ge_wn, preferred_element_type=jnp.float32)
                  + jnp.dot(max_f, ge_wx, preferred_element_type=jnp.float32)
                  + jnp.dot(zc, ge_wz, preferred_element_type=jnp.float32)
                  + out_b)
        feats.append(zc)
        cur = zc

    h = jnp.zeros((NPG, 64), jnp.float32)
    for i in range(4):
        h = h + jnp.dot(feats[i], nxt(), preferred_element_type=jnp.float32)
    h = _elu(h + nxt())
    for _ in range(3):
        h = _elu(jnp.dot(h, nxt(), preferred_element_type=jnp.float32) + nxt())
    h = _elu(jnp.dot(h, nxt(), preferred_element_type=jnp.float32) + nxt())
    h = _elu(jnp.dot(h, nxt(), preferred_element_type=jnp.float32) + nxt())
    h = jnp.dot(h, nxt(), preferred_element_type=jnp.float32) + nxt()
    oc = jnp.dot(h, nxt(), preferred_element_type=jnp.float32) + nxt()
    ob = jnp.dot(h, nxt(), preferred_element_type=jnp.float32) + nxt()
    out_ref[...] = jnp.concatenate([oc, ob], axis=1)


def _row(b):
    return b.reshape(1, -1)


def _flatten_params(params):
    ws = [params['input_w']]
    for i in range(4):
        p = params['blocks'][i]
        lw = p['lin_out_w']
        pw = p['post_w1']
        ow = p['out_w']
        ws += [
            p['pre_w1'], _row(p['pre_b1']), p['pre_w2'], _row(p['pre_b2']),
            p['lin_s_w'], _row(p['lin_s_b']), p['lin_h_w'], _row(p['lin_h_b']),
            lw[:DSH], lw[DSH:DSH + PROP], lw[DSH + PROP:], _row(p['lin_out_b']),
            pw[:DSH], pw[DSH:], _row(p['post_b1']),
            p['post_w2'], _row(p['post_b2']),
            ow[:DSH], ow[DSH:2 * DSH], ow[2 * DSH:3 * DSH], ow[3 * DSH:],
            _row(p['out_b']),
        ]
    pg0_w, pg0_b = params['postgn'][0]
    ws += [pg0_w[:DSH], pg0_w[DSH:2 * DSH], pg0_w[2 * DSH:3 * DSH],
           pg0_w[3 * DSH:], _row(pg0_b)]
    for i in range(1, 4):
        w, b = params['postgn'][i]
        ws += [w, _row(b)]
    for k in ('out1', 'out2', 'out3', 'clust', 'beta'):
        w, b = params[k]
        ws += [w, _row(b)]
    return ws


@jax.jit
def kernel(x, batch, params):
    ws = _flatten_params(params)
    in_specs = [pl.BlockSpec((NPG, IN_DIM), lambda e: (e, 0))]
    for w in ws:
        in_specs.append(pl.BlockSpec(w.shape, lambda e: (0, 0)))
    return pl.pallas_call(
        _gravnet_body,
        grid=(B,),
        in_specs=in_specs,
        out_specs=pl.BlockSpec((NPG, OUT_DIM), lambda e: (e, 0)),
        out_shape=jax.ShapeDtypeStruct((N, OUT_DIM), jnp.float32),
        compiler_params=pltpu.CompilerParams(
            dimension_semantics=("parallel",)),
    )(x, *ws)


# speedup vs baseline: 19.2434x; 1.4530x over previous
import functools

import jax
import jax.numpy as jnp
from jax.experimental import pallas as pl
from jax.experimental.pallas import tpu as pltpu

N = 10000
B = 10
NPG = N // B
K = 7
DSH = 32
PROP = 64
SDIM = 3
IN_DIM = 9
OUT_DIM = 31


def _elu(v):
    return jnp.where(v > 0, v, jnp.exp(v) - 1.0)


def _gravnet_body(x_ref, *refs):
    out_ref = refs[-1]
    wrefs = list(refs[:-1])
    it = iter(wrefs)

    def nxt():
        return next(it)[...]

    input_w = nxt()
    xb = x_ref[...]
    cur = jnp.dot(xb, input_w, preferred_element_type=jnp.float32)

    eye3 =(jax.lax.broadcasted_iota(jnp.int32, (SDIM, SDIM), 0)
            == jax.lax.broadcasted_iota(jnp.int32, (SDIM, SDIM), 1)
            ).astype(jnp.float32)

    feats = []
    for i in range(4):
        pre_w1 = nxt(); pre_b1 = nxt(); pre_w2 = nxt(); pre_b2 = nxt()
        lin_s_w = nxt(); lin_s_b = nxt(); lin_h_w = nxt(); lin_h_b = nxt()
        w_z = nxt(); w_mean = nxt(); w_max = nxt(); lin_out_b = nxt()
        post_wc = nxt(); post_ws = nxt(); post_b1 = nxt()
        post_w2 = nxt(); post_b2 = nxt()
        ge_wm = nxt(); ge_wn = nxt(); ge_wx = nxt(); ge_wz = nxt()
        out_b = nxt()

        z = _elu(jnp.dot(cur, pre_w1, preferred_element_type=jnp.float32) + pre_b1)
        z = _elu(jnp.dot(z, pre_w2, preferred_element_type=jnp.float32) + pre_b2)
        s = jnp.dot(z, lin_s_w, preferred_element_type=jnp.float32) + lin_s_b
        hf = jnp.dot(z, lin_h_w, preferred_element_type=jnp.float32) + lin_h_b

        sT = jax.lax.dot_general(eye3, s, (((1,), (1,)), ((), ())),
                                 preferred_element_type=jnp.float32)
        d2 = jnp.zeros((NPG, NPG), jnp.float32)
        for d in range(SDIM):
            diff = s[:, d:d + 1] - sT[d:d + 1, :]
            d2 = d2 + diff * diff

        def sel_step(_, carry):
            prev, accsum, accmax = carry
            dmin = jnp.min(jnp.where(d2 > prev, d2, jnp.float32(jnp.inf)),
                           axis=1, keepdims=True)
            selb = d2 == dmin
            g = jnp.dot(selb.astype(jnp.float32), hf,
                        preferred_element_type=jnp.float32)
            msg = g * jnp.exp(-10.0 * dmin)
            accsum = accsum + msg
            accmax = jnp.maximum(accmax, msg)
            return dmin, accsum, accmax

        _, accsum, accmax = jax.lax.fori_loop(
            0, K, sel_step,
            (jnp.full((NPG, 1), -jnp.inf, jnp.float32),
             jnp.zeros((NPG, PROP), jnp.float32),
             jnp.full((NPG, PROP), -jnp.inf, jnp.float32)))

        conv = (jnp.dot(z, w_z, preferred_element_type=jnp.float32)
                + jnp.dot(accsum * (1.0 / K), w_mean,
                          preferred_element_type=jnp.float32)
                + jnp.dot(accmax, w_max, preferred_element_type=jnp.float32)
                + lin_out_b)
        zc = _elu(jnp.dot(conv, post_wc, preferred_element_type=jnp.float32)
                  + jnp.dot(s, post_ws, preferred_element_type=jnp.float32)
                  + post_b1)
        zc = _elu(jnp.dot(zc, post_w2, preferred_element_type=jnp.float32)
                  + post_b2)

        mean_f = jnp.mean(zc, axis=0, keepdims=True)
        min_f = jnp.min(zc, axis=0, keepdims=True)
        max_f = jnp.max(zc, axis=0, keepdims=True)
        zc = _elu(jnp.dot(mean_f, ge_wm, preferred_element_type=jnp.float32)
                  + jnp.dot(min_f, ge_wn, preferred_element_type=jnp.float32)
                  + jnp.dot(max_f, ge_wx, preferred_element_type=jnp.float32)
                  + jnp.dot(zc, ge_wz, preferred_element_type=jnp.float32)
                  + out_b)
        feats.append(zc)
        cur = zc

    h = jnp.zeros((NPG, 64), jnp.float32)
    for i in range(4):
        h = h + jnp.dot(feats[i], nxt(), preferred_element_type=jnp.float32)
    h = _elu(h + nxt())
    for _ in range(3):
        h = _elu(jnp.dot(h, nxt(), preferred_element_type=jnp.float32) + nxt())
    h = _elu(jnp.dot(h, nxt(), preferred_element_type=jnp.float32) + nxt())
    h = _elu(jnp.dot(h, nxt(), preferred_element_type=jnp.float32) + nxt())
    h = jnp.dot(h, nxt(), preferred_element_type=jnp.float32) + nxt()
    oc = jnp.dot(h, nxt(), preferred_element_type=jnp.float32) + nxt()
    ob = jnp.dot(h, nxt(), preferred_element_type=jnp.float32) + nxt()
    out_ref[...] = jnp.concatenate([oc, ob], axis=1)


def _row(b):
    return b.reshape(1, -1)


def _flatten_params(params):
    ws = [params['input_w']]
    for i in range(4):
        p = params['blocks'][i]
        lw = p['lin_out_w']
        pw = p['post_w1']
        ow = p['out_w']
        ws += [
            p['pre_w1'], _row(p['pre_b1']), p['pre_w2'], _row(p['pre_b2']),
            p['lin_s_w'], _row(p['lin_s_b']), p['lin_h_w'], _row(p['lin_h_b']),
            lw[:DSH], lw[DSH:DSH + PROP], lw[DSH + PROP:], _row(p['lin_out_b']),
            pw[:DSH], pw[DSH:], _row(p['post_b1']),
            p['post_w2'], _row(p['post_b2']),
            ow[:DSH], ow[DSH:2 * DSH], ow[2 * DSH:3 * DSH], ow[3 * DSH:],
            _row(p['out_b']),
        ]
    pg0_w, pg0_b = params['postgn'][0]
    ws += [pg0_w[:DSH], pg0_w[DSH:2 * DSH], pg0_w[2 * DSH:3 * DSH],
           pg0_w[3 * DSH:], _row(pg0_b)]
    for i in range(1, 4):
        w, b = params['postgn'][i]
        ws += [w, _row(b)]
    for k in ('out1', 'out2', 'out3', 'clust', 'beta'):
        w, b = params[k]
        ws += [w, _row(b)]
    return ws


@jax.jit
def kernel(x, batch, params):
    ws = _flatten_params(params)
    in_specs = [pl.BlockSpec((NPG, IN_DIM), lambda e: (e, 0))]
    for w in ws:
        in_specs.append(pl.BlockSpec(w.shape, lambda e: (0, 0)))
    return pl.pallas_call(
        _gravnet_body,
        grid=(B,),
        in_specs=in_specs,
        out_specs=pl.BlockSpec((NPG, OUT_DIM), lambda e: (e, 0)),
        out_shape=jax.ShapeDtypeStruct((N, OUT_DIM), jnp.float32),
        compiler_params=pltpu.CompilerParams(
            dimension_semantics=("parallel",)),
    )(x, *ws)
